# baseline (speedup 1.0000x reference)
import jax
import jax.numpy as jnp
from jax.experimental import pallas as pl
from jax.experimental.pallas import tpu as pltpu

D_IN_ = 256
D_SAE_ = 4096
T_ = 32
K_ = 32
B_ = 4
M_ = B_ * T_
KC_ = T_ * D_IN_

BK_ = 512
BS_ = 2048
NK_ = KC_ // BK_
NS_ = D_SAE_ // BS_
OPB_ = BK_ // D_IN_

NEG_ = float("-inf")


def _fused_body(xcat_ref, w_ref, benc_ref, x_ref, wdec_ref, bdec_ref,
                z_ref, xhat_ref, loss_ref, xbig_ref, acc_ref, cand_ref):
    s = pl.program_id(0)
    k = pl.program_id(1)

    @pl.when((s == 0) & (k == 0))
    def _build():
        for o in range(T_):
            pieces = [xcat_ref[b * 2 * T_ + T_ - o: b * 2 * T_ + 2 * T_ - o, :]
                      for b in range(B_)]
            blk = o // OPB_
            col = (o % OPB_) * D_IN_
            xbig_ref[blk, :, col:col + D_IN_] = jnp.concatenate(pieces, axis=0)

    part = jnp.dot(xbig_ref[k], w_ref[...], preferred_element_type=jnp.float32)

    @pl.when(k == 0)
    def _init():
        acc_ref[s] = part

    @pl.when(k > 0)
    def _acc():
        acc_ref[s] += part

    @pl.when(k == NK_ - 1)
    def _select():
        pre_s = acc_ref[s] + benc_ref[...]
        acc_ref[s] = pre_s
        work = pre_s
        maxes = []
        for i in range(K_):
            m = jnp.max(work, axis=1, keepdims=True)
            maxes.append(m)
            if i < K_ - 1:
                work = jnp.where(work >= m, NEG_, work)
        cand_ref[s] = jnp.concatenate(maxes, axis=1)

    @pl.when((s == NS_ - 1) & (k == NK_ - 1))
    def _finish():
        cwork = cand_ref[...]
        for _ in range(K_ - 1):
            m = jnp.max(jnp.max(cwork, axis=0), axis=1, keepdims=True)
            cwork = jnp.where(cwork >= m[None], NEG_, cwork)
        thr = jnp.max(jnp.max(cwork, axis=0), axis=1, keepdims=True)
        pre = jnp.concatenate([acc_ref[i] for i in range(NS_)], axis=1)
        z = jnp.where(pre >= thr, jnp.maximum(pre, 0.0), 0.0)
        z_ref[...] = z
        xhat = (jnp.dot(z, wdec_ref[...], preferred_element_type=jnp.float32)
                + bdec_ref[...])
        xhat_ref[...] = xhat
        d = xhat - x_ref[...]
        loss_ref[0, 0] = jnp.sum(d * d) * (1.0 / M_)


@jax.jit
def kernel(x, W_enc_kernel, W_dec, b_enc, b_dec):
    xcat = jnp.pad(x, ((0, 0), (T_, 0), (0, 0))).reshape(B_ * 2 * T_, D_IN_)
    wbig = W_enc_kernel.reshape(KC_, D_SAE_)
    x2 = x.reshape(M_, D_IN_)

    z2, xhat2, loss2 = pl.pallas_call(
        _fused_body,
        grid=(NS_, NK_),
        in_specs=[
            pl.BlockSpec((B_ * 2 * T_, D_IN_), lambda s, k: (0, 0)),
            pl.BlockSpec((BK_, BS_), lambda s, k: (k, s)),
            pl.BlockSpec((1, BS_), lambda s, k: (0, s)),
            pl.BlockSpec((M_, D_IN_), lambda s, k: (0, 0)),
            pl.BlockSpec((D_SAE_, D_IN_), lambda s, k: (0, 0)),
            pl.BlockSpec((1, D_IN_), lambda s, k: (0, 0)),
        ],
        out_specs=[
            pl.BlockSpec((M_, D_SAE_), lambda s, k: (0, 0)),
            pl.BlockSpec((M_, D_IN_), lambda s, k: (0, 0)),
            pl.BlockSpec(memory_space=pltpu.SMEM),
        ],
        out_shape=[
            jax.ShapeDtypeStruct((M_, D_SAE_), jnp.float32),
            jax.ShapeDtypeStruct((M_, D_IN_), jnp.float32),
            jax.ShapeDtypeStruct((1, 1), jnp.float32),
        ],
        scratch_shapes=[
            pltpu.VMEM((NK_, M_, BK_), jnp.float32),
            pltpu.VMEM((NS_, M_, BS_), jnp.float32),
            pltpu.VMEM((NS_, M_, K_), jnp.float32),
        ],
    )(xcat, wbig, b_enc.reshape(1, D_SAE_), x2, W_dec,
      b_dec.reshape(1, D_IN_))

    z = z2.reshape(B_, T_, D_SAE_)
    x_hat = xhat2.reshape(B_, T_, D_IN_)
    loss = loss2[0, 0]
    return (loss, x_hat, z)


# speedup vs baseline: 1.2242x; 1.2242x over previous
import jax
import jax.numpy as jnp
from jax.experimental import pallas as pl
from jax.experimental.pallas import tpu as pltpu

D_IN_ = 256
D_SAE_ = 4096
T_ = 32
K_ = 32
B_ = 4
M_ = B_ * T_
KC_ = T_ * D_IN_

BK_ = 512
NK_ = KC_ // BK_
OPB_ = BK_ // D_IN_

NEG_ = float("-inf")


def _fused_body(xcat_ref, w_ref, benc_ref, x_ref, wdec_ref, bdec_ref,
                z_ref, xhat_ref, loss_ref, xbig_ref, acc_ref):
    k = pl.program_id(0)

    @pl.when(k == 0)
    def _build():
        for o in range(T_):
            pieces = [xcat_ref[b * 2 * T_ + T_ - o: b * 2 * T_ + 2 * T_ - o, :]
                      for b in range(B_)]
            blk = o // OPB_
            col = (o % OPB_) * D_IN_
            xbig_ref[blk, :, col:col + D_IN_] = jnp.concatenate(pieces, axis=0)

    part = jnp.dot(xbig_ref[k], w_ref[...], preferred_element_type=jnp.float32)

    @pl.when(k == 0)
    def _init():
        acc_ref[...] = part

    @pl.when(k > 0)
    def _acc():
        acc_ref[...] += part

    @pl.when(k == NK_ - 1)
    def _finish():
        pre = acc_ref[...] + benc_ref[...]
        work = pre
        for _ in range(K_ - 1):
            m = jnp.max(work, axis=1, keepdims=True)
            work = jnp.where(work >= m, NEG_, work)
        thr = jnp.max(work, axis=1, keepdims=True)
        z = jnp.where(pre >= thr, jnp.maximum(pre, 0.0), 0.0)
        z_ref[...] = z
        xhat = (jnp.dot(z, wdec_ref[...], preferred_element_type=jnp.float32)
                + bdec_ref[...])
        xhat_ref[...] = xhat
        d = xhat - x_ref[...]
        loss_ref[0, 0] = jnp.sum(d * d) * (1.0 / M_)


@jax.jit
def kernel(x, W_enc_kernel, W_dec, b_enc, b_dec):
    xcat = jnp.pad(x, ((0, 0), (T_, 0), (0, 0))).reshape(B_ * 2 * T_, D_IN_)
    wbig = W_enc_kernel.reshape(KC_, D_SAE_)
    x2 = x.reshape(M_, D_IN_)

    z2, xhat2, loss2 = pl.pallas_call(
        _fused_body,
        grid=(NK_,),
        in_specs=[
            pl.BlockSpec((B_ * 2 * T_, D_IN_), lambda k: (0, 0)),
            pl.BlockSpec((BK_, D_SAE_), lambda k: (k, 0)),
            pl.BlockSpec((1, D_SAE_), lambda k: (0, 0)),
            pl.BlockSpec((M_, D_IN_), lambda k: (0, 0)),
            pl.BlockSpec((D_SAE_, D_IN_), lambda k: (0, 0)),
            pl.BlockSpec((1, D_IN_), lambda k: (0, 0)),
        ],
        out_specs=[
            pl.BlockSpec((M_, D_SAE_), lambda k: (0, 0)),
            pl.BlockSpec((M_, D_IN_), lambda k: (0, 0)),
            pl.BlockSpec(memory_space=pltpu.SMEM),
        ],
        out_shape=[
            jax.ShapeDtypeStruct((M_, D_SAE_), jnp.float32),
            jax.ShapeDtypeStruct((M_, D_IN_), jnp.float32),
            jax.ShapeDtypeStruct((1, 1), jnp.float32),
        ],
        scratch_shapes=[
            pltpu.VMEM((NK_, M_, BK_), jnp.float32),
            pltpu.VMEM((M_, D_SAE_), jnp.float32),
        ],
    )(xcat, wbig, b_enc.reshape(1, D_SAE_), x2, W_dec,
      b_dec.reshape(1, D_IN_))

    z = z2.reshape(B_, T_, D_SAE_)
    x_hat = xhat2.reshape(B_, T_, D_IN_)
    loss = loss2[0, 0]
    return (loss, x_hat, z)
